# Initial kernel scaffold; baseline (speedup 1.0000x reference)
import jax
import jax.numpy as jnp
from jax.experimental import pallas as pl


def kernel(x, edge_index, W1l, b1l, W1r, W2l, b2l, W2r, Wm1, bm1, Wm2, bm2):
    raise NotImplementedError("write your pallas kernel here")


# speedup vs baseline: 3.2558x; 3.2558x over previous
import functools

import jax
import jax.numpy as jnp
from jax import lax
from jax.experimental import pallas as pl
from jax.experimental.pallas import tpu as pltpu
from jax.experimental.pallas import tpu_sc as plsc

N = 10000
NP = 10240
E = 160000
D_IN = 256
D_HID = 512
FC = 128
NTILES = 16
ROWS_PT = NP // NTILES
EPT = E // NTILES
BLK = 80
NBLK = EPT // BLK
EPT_C = E // 32
BLK_C = 40
NBLK_C = EPT_C // BLK_C
ZR = 80

_mesh = plsc.VectorSubcoreMesh(core_axis_name="c", subcore_axis_name="s")


@functools.partial(
    pl.kernel,
    mesh=_mesh,
    out_type=jax.ShapeDtypeStruct((NP, FC), jnp.float32),
    scratch_types=[
        pltpu.VMEM((NBLK, BLK), jnp.int32),
        pltpu.VMEM((BLK, FC), jnp.float32),
        pltpu.VMEM_SHARED((NP, FC), jnp.float32),
    ],
)
def _sc_degree(dst3, zrow, ones_h, deg_out, didx_v, ones_v, deg_sh):
    c = lax.axis_index("c")
    s = lax.axis_index("s")
    pltpu.sync_copy(dst3.at[s], didx_v)
    pltpu.sync_copy(ones_h, ones_v)
    for z in range(ROWS_PT // ZR):
        pltpu.sync_copy(zrow, deg_sh.at[pl.ds(s * ROWS_PT + z * ZR, ZR)])
    plsc.subcore_barrier()

    def body(j, carry):
        @pl.when(c == 0)
        def _():
            pltpu.sync_copy(ones_v, deg_sh.at[didx_v.at[j]], add=True)
        return carry

    lax.fori_loop(0, NBLK, body, 0)
    plsc.subcore_barrier()

    @pl.when(c == 0)
    def _():
        pltpu.sync_copy(deg_sh.at[pl.ds(s * ROWS_PT, ROWS_PT)],
                        deg_out.at[pl.ds(s * ROWS_PT, ROWS_PT)])


@functools.partial(
    pl.kernel,
    mesh=_mesh,
    out_type=jax.ShapeDtypeStruct((2, NP, FC), jnp.float32),
    scratch_types=[
        pltpu.VMEM((NBLK, BLK), jnp.int32),
        pltpu.VMEM((NBLK, BLK), jnp.int32),
        pltpu.VMEM((BLK, FC), jnp.float32),
        pltpu.VMEM_SHARED((NP, FC), jnp.float32),
        pltpu.SemaphoreType.DMA,
    ],
)
def _sc_layer1(xa, xb, src3, dst3, zrow,
               agg_out, sidx_v, didx_v, rows_v, acc_sh, sem):
    c = lax.axis_index("c")
    s = lax.axis_index("s")
    pltpu.sync_copy(src3.at[s], sidx_v)
    pltpu.sync_copy(dst3.at[s], didx_v)
    for z in range(ROWS_PT // ZR):
        pltpu.sync_copy(zrow, acc_sh.at[pl.ds(s * ROWS_PT + z * ZR, ZR)])
    plsc.subcore_barrier()

    def body(j, carry):
        @pl.when(c == 0)
        def _():
            pltpu.async_copy(xa.at[sidx_v.at[j]], rows_v, sem).wait()

        @pl.when(c == 1)
        def _():
            pltpu.async_copy(xb.at[sidx_v.at[j]], rows_v, sem).wait()

        pltpu.sync_copy(rows_v, acc_sh.at[didx_v.at[j]], add=True)
        return carry

    lax.fori_loop(0, NBLK, body, 0)
    plsc.subcore_barrier()
    pltpu.sync_copy(acc_sh.at[pl.ds(s * ROWS_PT, ROWS_PT)],
                    agg_out.at[c, pl.ds(s * ROWS_PT, ROWS_PT)])


@functools.partial(
    pl.kernel,
    mesh=_mesh,
    out_type=jax.ShapeDtypeStruct((2, 2, NP, FC), jnp.float32),
    scratch_types=[
        pltpu.VMEM((NBLK, BLK), jnp.int32),
        pltpu.VMEM((NBLK, BLK), jnp.int32),
        pltpu.VMEM((BLK, FC), jnp.float32),
        pltpu.VMEM_SHARED((NP, FC), jnp.float32),
        pltpu.SemaphoreType.DMA,
    ],
)
def _sc_layer2(h0, h1c, h2c, h3c, src3, dst3, zrow,
               agg_out, sidx_v, didx_v, rows_v, acc_sh, sem):
    c = lax.axis_index("c")
    s = lax.axis_index("s")
    pltpu.sync_copy(src3.at[s], sidx_v)
    pltpu.sync_copy(dst3.at[s], didx_v)
    chunks = ((h0, h2c), (h1c, h3c))
    for sub in (0, 1):
        for z in range(ROWS_PT // ZR):
            pltpu.sync_copy(zrow, acc_sh.at[pl.ds(s * ROWS_PT + z * ZR, ZR)])
        plsc.subcore_barrier()
        ca, cb = chunks[sub]

        def body(j, carry):
            @pl.when(c == 0)
            def _():
                pltpu.async_copy(ca.at[sidx_v.at[j]], rows_v, sem).wait()

            @pl.when(c == 1)
            def _():
                pltpu.async_copy(cb.at[sidx_v.at[j]], rows_v, sem).wait()

            pltpu.sync_copy(rows_v, acc_sh.at[didx_v.at[j]], add=True)
            return carry

        lax.fori_loop(0, NBLK, body, 0)
        plsc.subcore_barrier()
        pltpu.sync_copy(acc_sh.at[pl.ds(s * ROWS_PT, ROWS_PT)],
                        agg_out.at[c, sub, pl.ds(s * ROWS_PT, ROWS_PT)])


@functools.partial(
    pl.kernel,
    mesh=_mesh,
    out_type=[
        jax.ShapeDtypeStruct((E, FC), jnp.float32),
        jax.ShapeDtypeStruct((E, FC), jnp.float32),
    ],
    scratch_types=[
        pltpu.VMEM((NBLK_C, BLK_C), jnp.int32),
        pltpu.VMEM((NBLK_C, BLK_C), jnp.int32),
        pltpu.VMEM((BLK_C, FC), jnp.float32),
        pltpu.VMEM((BLK_C, FC), jnp.float32),
        pltpu.SemaphoreType.DMA,
    ],
)
def _sc_edge(pa, pb, src3, dst3, a_out, b_out,
             sidx_v, didx_v, rowsa_v, rowsb_v, sem):
    c = lax.axis_index("c")
    s = lax.axis_index("s")
    wid = s * 2 + c
    pltpu.sync_copy(src3.at[wid], sidx_v)
    pltpu.sync_copy(dst3.at[wid], didx_v)

    def body(j, carry):
        pltpu.async_copy(pa.at[sidx_v.at[j]], rowsa_v, sem).wait()
        pltpu.async_copy(pb.at[didx_v.at[j]], rowsb_v, sem).wait()
        base = wid * EPT_C + j * BLK_C
        pltpu.sync_copy(rowsa_v, a_out.at[pl.ds(base, BLK_C)])
        pltpu.sync_copy(rowsb_v, b_out.at[pl.ds(base, BLK_C)])
        return carry

    lax.fori_loop(0, NBLK_C, body, 0)


def _tc1_fn(agg_ref, deg_ref, x_ref, wl_ref, bl_ref, wr_ref, o_ref):
    inv = 1.0 / jnp.maximum(deg_ref[...], 1.0)
    mean = agg_ref[...] * inv
    acc = jnp.dot(mean, wl_ref[...], preferred_element_type=jnp.float32)
    acc = acc + jnp.dot(x_ref[...], wr_ref[...],
                        preferred_element_type=jnp.float32)
    o_ref[...] = jnp.maximum(acc + bl_ref[...], 0.0)


_BM1 = 1000
_tc1 = pl.pallas_call(
    _tc1_fn,
    grid=(N // _BM1,),
    in_specs=[
        pl.BlockSpec((_BM1, D_IN), lambda i: (i, 0)),
        pl.BlockSpec((_BM1, 1), lambda i: (i, 0)),
        pl.BlockSpec((_BM1, D_IN), lambda i: (i, 0)),
        pl.BlockSpec((D_IN, D_HID), lambda i: (0, 0)),
        pl.BlockSpec((1, D_HID), lambda i: (0, 0)),
        pl.BlockSpec((D_IN, D_HID), lambda i: (0, 0)),
    ],
    out_specs=pl.BlockSpec((_BM1, D_HID), lambda i: (i, 0)),
    out_shape=jax.ShapeDtypeStruct((N, D_HID), jnp.float32),
)


def _tc2_fn(agg_ref, deg_ref, h_ref, wl_ref, bl_ref, wr_ref,
            wma_ref, wmb_ref, bm1_ref, pa_ref, pb_ref):
    inv = 1.0 / jnp.maximum(deg_ref[...], 1.0)
    mean = agg_ref[...] * inv
    h2 = jnp.dot(mean, wl_ref[...], preferred_element_type=jnp.float32)
    h2 = h2 + jnp.dot(h_ref[...], wr_ref[...],
                      preferred_element_type=jnp.float32)
    h2 = h2 + bl_ref[...]
    pa_ref[...] = jnp.dot(h2, wma_ref[...],
                          preferred_element_type=jnp.float32) + bm1_ref[...]
    pb_ref[...] = jnp.dot(h2, wmb_ref[...],
                          preferred_element_type=jnp.float32)


_tc2 = pl.pallas_call(
    _tc2_fn,
    grid=(N // _BM1,),
    in_specs=[
        pl.BlockSpec((_BM1, D_HID), lambda i: (i, 0)),
        pl.BlockSpec((_BM1, 1), lambda i: (i, 0)),
        pl.BlockSpec((_BM1, D_HID), lambda i: (i, 0)),
        pl.BlockSpec((D_HID, D_HID), lambda i: (0, 0)),
        pl.BlockSpec((1, D_HID), lambda i: (0, 0)),
        pl.BlockSpec((D_HID, D_HID), lambda i: (0, 0)),
        pl.BlockSpec((D_HID, FC), lambda i: (0, 0)),
        pl.BlockSpec((D_HID, FC), lambda i: (0, 0)),
        pl.BlockSpec((1, FC), lambda i: (0, 0)),
    ],
    out_specs=[
        pl.BlockSpec((_BM1, FC), lambda i: (i, 0)),
        pl.BlockSpec((_BM1, FC), lambda i: (i, 0)),
    ],
    out_shape=[
        jax.ShapeDtypeStruct((N, FC), jnp.float32),
        jax.ShapeDtypeStruct((N, FC), jnp.float32),
    ],
)


def _tc3_fn(a_ref, b_ref, w_ref, o_ref):
    t = jnp.maximum(a_ref[...] + b_ref[...], 0.0)
    o_ref[...] = lax.dot_general(t, w_ref[...], (((1,), (1,)), ((), ())),
                                 preferred_element_type=jnp.float32)


_BM3 = 4000
_tc3 = pl.pallas_call(
    _tc3_fn,
    grid=(E // _BM3,),
    in_specs=[
        pl.BlockSpec((_BM3, FC), lambda i: (i, 0)),
        pl.BlockSpec((_BM3, FC), lambda i: (i, 0)),
        pl.BlockSpec((1, FC), lambda i: (0, 0)),
    ],
    out_specs=pl.BlockSpec((_BM3, 1), lambda i: (i, 0)),
    out_shape=jax.ShapeDtypeStruct((E, 1), jnp.float32),
)


def kernel(x, edge_index, W1l, b1l, W1r, W2l, b2l, W2r, Wm1, bm1, Wm2, bm2):
    src = edge_index[0].astype(jnp.int32)
    dst = edge_index[1].astype(jnp.int32)
    src3 = src.reshape(NTILES, NBLK, BLK)
    dst3 = dst.reshape(NTILES, NBLK, BLK)
    src3c = src.reshape(32, NBLK_C, BLK_C)
    dst3c = dst.reshape(32, NBLK_C, BLK_C)

    zrow = jnp.zeros((ZR, FC), jnp.float32)
    zdeg = jnp.zeros((ROWS_PT, 16), jnp.float32)
    ones_h = jnp.ones((BLK, 16), jnp.float32)

    deg = jax.ops.segment_sum(jnp.ones((E,), jnp.float32), dst,
                              num_segments=N).reshape(N, 1)

    agg = _sc_layer1(x[:, :FC], x[:, FC:], src3, dst3, zrow)
    agg1 = jnp.concatenate([agg[0, :N], agg[1, :N]], axis=1)

    h1 = _tc1(agg1, deg, x, W1l.T, b1l.reshape(1, -1), W1r.T)

    agg2c = _sc_layer2(h1[:, :FC], h1[:, FC:2 * FC], h1[:, 2 * FC:3 * FC],
                       h1[:, 3 * FC:], src3, dst3, zrow)
    agg2 = jnp.concatenate([agg2c[0, 0, :N], agg2c[0, 1, :N],
                            agg2c[1, 0, :N], agg2c[1, 1, :N]], axis=1)

    pa, pb = _tc2(agg2, deg, h1, W2l.T, b2l.reshape(1, -1), W2r.T,
                  Wm1[:, :D_HID].T, Wm1[:, D_HID:].T, bm1.reshape(1, -1))

    a_rows, b_rows = _sc_edge(pa, pb, src3c, dst3c)

    out = _tc3(a_rows, b_rows, Wm2.reshape(1, -1))
    return out[:, 0] + bm2[0]


# speedup vs baseline: 3.7876x; 1.1633x over previous
import functools

import jax
import jax.numpy as jnp
from jax import lax
from jax.experimental import pallas as pl
from jax.experimental.pallas import tpu as pltpu
from jax.experimental.pallas import tpu_sc as plsc

N = 10000
NP = 10240
E = 160000
D_IN = 256
D_HID = 512
FC = 128
NTILES = 16
ROWS_PT = NP // NTILES
EPT = E // NTILES
BLK = 80
NBLK = EPT // BLK
EPT_C = E // 32
BLK_C = 40
NBLK_C = EPT_C // BLK_C
ZR = 80

_mesh = plsc.VectorSubcoreMesh(core_axis_name="c", subcore_axis_name="s")


@functools.partial(
    pl.kernel,
    mesh=_mesh,
    out_type=jax.ShapeDtypeStruct((NP, 16), jnp.float32),
    scratch_types=[
        pltpu.VMEM((NBLK, BLK), jnp.int32),
        pltpu.VMEM((BLK, 16), jnp.float32),
        pltpu.VMEM_SHARED((NP, 16), jnp.float32),
    ],
)
def _sc_degree(dst3, zdeg, ones_h, deg_out, didx_v, ones_v, deg_sh):
    c = lax.axis_index("c")
    s = lax.axis_index("s")
    pltpu.sync_copy(dst3.at[s], didx_v)
    pltpu.sync_copy(ones_h, ones_v)
    pltpu.sync_copy(zdeg, deg_sh.at[pl.ds(s * ROWS_PT, ROWS_PT)])
    plsc.subcore_barrier()

    def body(j, carry):
        @pl.when(c == 0)
        def _():
            pltpu.sync_copy(ones_v, deg_sh.at[didx_v.at[j]], add=True)
        return carry

    lax.fori_loop(0, NBLK, body, 0)
    plsc.subcore_barrier()

    @pl.when(c == 0)
    def _():
        pltpu.sync_copy(deg_sh.at[pl.ds(s * ROWS_PT, ROWS_PT)],
                        deg_out.at[pl.ds(s * ROWS_PT, ROWS_PT)])


@functools.partial(
    pl.kernel,
    mesh=_mesh,
    out_type=jax.ShapeDtypeStruct((2, NP, FC), jnp.float32),
    scratch_types=[
        pltpu.VMEM((NBLK, BLK), jnp.int32),
        pltpu.VMEM((NBLK, BLK), jnp.int32),
        pltpu.VMEM((BLK, FC), jnp.float32),
        pltpu.VMEM_SHARED((NP, FC), jnp.float32),
        pltpu.SemaphoreType.DMA,
    ],
)
def _sc_layer1(xa, xb, src3, dst3, zrow,
               agg_out, sidx_v, didx_v, rows_v, acc_sh, sem):
    c = lax.axis_index("c")
    s = lax.axis_index("s")
    pltpu.sync_copy(src3.at[s], sidx_v)
    pltpu.sync_copy(dst3.at[s], didx_v)
    for z in range(ROWS_PT // ZR):
        pltpu.sync_copy(zrow, acc_sh.at[pl.ds(s * ROWS_PT + z * ZR, ZR)])
    plsc.subcore_barrier()

    def body(j, carry):
        @pl.when(c == 0)
        def _():
            pltpu.async_copy(xa.at[sidx_v.at[j]], rows_v, sem).wait()

        @pl.when(c == 1)
        def _():
            pltpu.async_copy(xb.at[sidx_v.at[j]], rows_v, sem).wait()

        pltpu.sync_copy(rows_v, acc_sh.at[didx_v.at[j]], add=True)
        return carry

    lax.fori_loop(0, NBLK, body, 0)
    plsc.subcore_barrier()
    pltpu.sync_copy(acc_sh.at[pl.ds(s * ROWS_PT, ROWS_PT)],
                    agg_out.at[c, pl.ds(s * ROWS_PT, ROWS_PT)])


@functools.partial(
    pl.kernel,
    mesh=_mesh,
    out_type=jax.ShapeDtypeStruct((2, 2, NP, FC), jnp.float32),
    scratch_types=[
        pltpu.VMEM((NBLK, BLK), jnp.int32),
        pltpu.VMEM((NBLK, BLK), jnp.int32),
        pltpu.VMEM((BLK, FC), jnp.float32),
        pltpu.VMEM_SHARED((NP, FC), jnp.float32),
        pltpu.SemaphoreType.DMA,
    ],
)
def _sc_layer2(h0, h1c, h2c, h3c, src3, dst3, zrow,
               agg_out, sidx_v, didx_v, rows_v, acc_sh, sem):
    c = lax.axis_index("c")
    s = lax.axis_index("s")
    pltpu.sync_copy(src3.at[s], sidx_v)
    pltpu.sync_copy(dst3.at[s], didx_v)
    chunks = ((h0, h2c), (h1c, h3c))
    for sub in (0, 1):
        for z in range(ROWS_PT // ZR):
            pltpu.sync_copy(zrow, acc_sh.at[pl.ds(s * ROWS_PT + z * ZR, ZR)])
        plsc.subcore_barrier()
        ca, cb = chunks[sub]

        def body(j, carry):
            @pl.when(c == 0)
            def _():
                pltpu.async_copy(ca.at[sidx_v.at[j]], rows_v, sem).wait()

            @pl.when(c == 1)
            def _():
                pltpu.async_copy(cb.at[sidx_v.at[j]], rows_v, sem).wait()

            pltpu.sync_copy(rows_v, acc_sh.at[didx_v.at[j]], add=True)
            return carry

        lax.fori_loop(0, NBLK, body, 0)
        plsc.subcore_barrier()
        pltpu.sync_copy(acc_sh.at[pl.ds(s * ROWS_PT, ROWS_PT)],
                        agg_out.at[c, sub, pl.ds(s * ROWS_PT, ROWS_PT)])


@functools.partial(
    pl.kernel,
    mesh=_mesh,
    out_type=[
        jax.ShapeDtypeStruct((E, FC), jnp.float32),
        jax.ShapeDtypeStruct((E, FC), jnp.float32),
    ],
    scratch_types=[
        pltpu.VMEM((NBLK_C, BLK_C), jnp.int32),
        pltpu.VMEM((NBLK_C, BLK_C), jnp.int32),
        pltpu.VMEM((BLK_C, FC), jnp.float32),
        pltpu.VMEM((BLK_C, FC), jnp.float32),
        pltpu.SemaphoreType.DMA,
    ],
)
def _sc_edge(pa, pb, src3, dst3, a_out, b_out,
             sidx_v, didx_v, rowsa_v, rowsb_v, sem):
    c = lax.axis_index("c")
    s = lax.axis_index("s")
    wid = s * 2 + c
    pltpu.sync_copy(src3.at[wid], sidx_v)
    pltpu.sync_copy(dst3.at[wid], didx_v)

    def body(j, carry):
        pltpu.async_copy(pa.at[sidx_v.at[j]], rowsa_v, sem).wait()
        pltpu.async_copy(pb.at[didx_v.at[j]], rowsb_v, sem).wait()
        base = wid * EPT_C + j * BLK_C
        pltpu.sync_copy(rowsa_v, a_out.at[pl.ds(base, BLK_C)])
        pltpu.sync_copy(rowsb_v, b_out.at[pl.ds(base, BLK_C)])
        return carry

    lax.fori_loop(0, NBLK_C, body, 0)


def _tc1_fn(agg_ref, deg_ref, x_ref, wl_ref, bl_ref, wr_ref, o_ref):
    inv = 1.0 / jnp.maximum(deg_ref[...], 1.0)
    mean = agg_ref[...] * inv
    acc = jnp.dot(mean, wl_ref[...], preferred_element_type=jnp.float32)
    acc = acc + jnp.dot(x_ref[...], wr_ref[...],
                        preferred_element_type=jnp.float32)
    o_ref[...] = jnp.maximum(acc + bl_ref[...], 0.0)


_BM1 = 1000
_tc1 = pl.pallas_call(
    _tc1_fn,
    grid=(N // _BM1,),
    in_specs=[
        pl.BlockSpec((_BM1, D_IN), lambda i: (i, 0)),
        pl.BlockSpec((_BM1, 1), lambda i: (i, 0)),
        pl.BlockSpec((_BM1, D_IN), lambda i: (i, 0)),
        pl.BlockSpec((D_IN, D_HID), lambda i: (0, 0)),
        pl.BlockSpec((1, D_HID), lambda i: (0, 0)),
        pl.BlockSpec((D_IN, D_HID), lambda i: (0, 0)),
    ],
    out_specs=pl.BlockSpec((_BM1, D_HID), lambda i: (i, 0)),
    out_shape=jax.ShapeDtypeStruct((N, D_HID), jnp.float32),
)


def _tc2_fn(agg_ref, deg_ref, h_ref, wl_ref, bl_ref, wr_ref,
            wma_ref, wmb_ref, bm1_ref, pa_ref, pb_ref):
    inv = 1.0 / jnp.maximum(deg_ref[...], 1.0)
    mean = agg_ref[...] * inv
    h2 = jnp.dot(mean, wl_ref[...], preferred_element_type=jnp.float32)
    h2 = h2 + jnp.dot(h_ref[...], wr_ref[...],
                      preferred_element_type=jnp.float32)
    h2 = h2 + bl_ref[...]
    pa_ref[...] = jnp.dot(h2, wma_ref[...],
                          preferred_element_type=jnp.float32) + bm1_ref[...]
    pb_ref[...] = jnp.dot(h2, wmb_ref[...],
                          preferred_element_type=jnp.float32)


_tc2 = pl.pallas_call(
    _tc2_fn,
    grid=(N // _BM1,),
    in_specs=[
        pl.BlockSpec((_BM1, D_HID), lambda i: (i, 0)),
        pl.BlockSpec((_BM1, 1), lambda i: (i, 0)),
        pl.BlockSpec((_BM1, D_HID), lambda i: (i, 0)),
        pl.BlockSpec((D_HID, D_HID), lambda i: (0, 0)),
        pl.BlockSpec((1, D_HID), lambda i: (0, 0)),
        pl.BlockSpec((D_HID, D_HID), lambda i: (0, 0)),
        pl.BlockSpec((D_HID, FC), lambda i: (0, 0)),
        pl.BlockSpec((D_HID, FC), lambda i: (0, 0)),
        pl.BlockSpec((1, FC), lambda i: (0, 0)),
    ],
    out_specs=[
        pl.BlockSpec((_BM1, FC), lambda i: (i, 0)),
        pl.BlockSpec((_BM1, FC), lambda i: (i, 0)),
    ],
    out_shape=[
        jax.ShapeDtypeStruct((N, FC), jnp.float32),
        jax.ShapeDtypeStruct((N, FC), jnp.float32),
    ],
)


def _tc3_fn(a_ref, b_ref, w_ref, o_ref):
    t = jnp.maximum(a_ref[...] + b_ref[...], 0.0)
    o_ref[...] = lax.dot_general(t, w_ref[...], (((1,), (1,)), ((), ())),
                                 preferred_element_type=jnp.float32)


_BM3 = 4000
_tc3 = pl.pallas_call(
    _tc3_fn,
    grid=(E // _BM3,),
    in_specs=[
        pl.BlockSpec((_BM3, FC), lambda i: (i, 0)),
        pl.BlockSpec((_BM3, FC), lambda i: (i, 0)),
        pl.BlockSpec((1, FC), lambda i: (0, 0)),
    ],
    out_specs=pl.BlockSpec((_BM3, 1), lambda i: (i, 0)),
    out_shape=jax.ShapeDtypeStruct((E, 1), jnp.float32),
)


def kernel(x, edge_index, W1l, b1l, W1r, W2l, b2l, W2r, Wm1, bm1, Wm2, bm2):
    src = edge_index[0].astype(jnp.int32)
    dst = edge_index[1].astype(jnp.int32)
    src3 = src.reshape(NTILES, NBLK, BLK)
    dst3 = dst.reshape(NTILES, NBLK, BLK)
    src3c = src.reshape(32, NBLK_C, BLK_C)
    dst3c = dst.reshape(32, NBLK_C, BLK_C)

    zrow = jnp.zeros((ZR, FC), jnp.float32)
    zdeg = jnp.zeros((ROWS_PT, 16), jnp.float32)
    ones_h = jnp.ones((BLK, 16), jnp.float32)

    degc = _sc_degree(dst3, zdeg, ones_h)
    deg = degc[:N, :1]

    agg = _sc_layer1(x[:, :FC], x[:, FC:], src3, dst3, zrow)
    agg1 = jnp.concatenate([agg[0, :N], agg[1, :N]], axis=1)

    h1 = _tc1(agg1, deg, x, W1l.T, b1l.reshape(1, -1), W1r.T)

    agg2c = _sc_layer2(h1[:, :FC], h1[:, FC:2 * FC], h1[:, 2 * FC:3 * FC],
                       h1[:, 3 * FC:], src3, dst3, zrow)
    agg2 = jnp.concatenate([agg2c[0, 0, :N], agg2c[0, 1, :N],
                            agg2c[1, 0, :N], agg2c[1, 1, :N]], axis=1)

    pa, pb = _tc2(agg2, deg, h1, W2l.T, b2l.reshape(1, -1), W2r.T,
                  Wm1[:, :D_HID].T, Wm1[:, D_HID:].T, bm1.reshape(1, -1))

    a_rows, b_rows = _sc_edge(pa, pb, src3c, dst3c)

    out = _tc3(a_rows, b_rows, Wm2.reshape(1, -1))
    return out[:, 0] + bm2[0]
